# Initial kernel scaffold; baseline (speedup 1.0000x reference)
import jax
import jax.numpy as jnp
from jax.experimental import pallas as pl


def kernel(tokens, token_emb, pos_emb, W, b):
    raise NotImplementedError("write your pallas kernel here")


# speedup vs baseline: 2.9380x; 2.9380x over previous
import functools

import jax
import jax.numpy as jnp
from jax import lax
from jax.experimental import pallas as pl
from jax.experimental.pallas import tpu as pltpu
from jax.experimental.pallas import tpu_sc as plsc

_NC, _NS = 2, 16
_NW = _NC * _NS
_GATHER_CHUNK = 64


def _sc_gather(table, idx_flat):
    total, d = idx_flat.shape[0], table.shape[1]
    b_per_w = total // _NW
    n_chunks = b_per_w // _GATHER_CHUNK
    mesh = plsc.VectorSubcoreMesh(core_axis_name="c", subcore_axis_name="s")

    @functools.partial(
        pl.kernel,
        mesh=mesh,
        out_type=jax.ShapeDtypeStruct((total, d), table.dtype),
        scratch_types=[
            pltpu.VMEM((b_per_w,), jnp.int32),
            pltpu.VMEM((_GATHER_CHUNK, d), table.dtype),
            pltpu.SemaphoreType.DMA,
        ],
    )
    def gather_kernel(table_hbm, idx_hbm, out_hbm, idx_v, rows_v, sem):
        wid = lax.axis_index("s") * _NC + lax.axis_index("c")
        base = wid * b_per_w
        pltpu.sync_copy(idx_hbm.at[pl.ds(base, b_per_w)], idx_v)

        @pl.loop(0, n_chunks)
        def _(c):
            off = c * _GATHER_CHUNK
            pltpu.async_copy(
                table_hbm.at[idx_v.at[pl.ds(off, _GATHER_CHUNK)]], rows_v, sem
            ).wait()
            pltpu.sync_copy(rows_v, out_hbm.at[pl.ds(base + off, _GATHER_CHUNK)])

    return gather_kernel(table, idx_flat)


def _mlp_body(x_ref, p_ref, w_ref, b_ref, o_ref):
    h = (x_ref[...] + p_ref[...]).astype(jnp.bfloat16)
    acc = jnp.dot(h, w_ref[...], preferred_element_type=jnp.float32)
    acc = acc + b_ref[...]
    o_ref[...] = 0.5 * acc * (1.0 + lax.erf(acc * 0.7071067811865476))


def _tc_mlp(hidden, pos_emb, w_bf16, bias_2d, seq_len):
    m, k = hidden.shape
    n = w_bf16.shape[1]
    bm = 512
    pos_blocks = seq_len // bm
    grid = (m // bm,)

    return pl.pallas_call(
        _mlp_body,
        grid=grid,
        in_specs=[
            pl.BlockSpec((bm, k), lambda i: (i, 0)),
            pl.BlockSpec((bm, k), lambda i: (i % pos_blocks, 0)),
            pl.BlockSpec((k, n), lambda i: (0, 0)),
            pl.BlockSpec((1, n), lambda i: (0, 0)),
        ],
        out_specs=pl.BlockSpec((bm, n), lambda i: (i, 0)),
        out_shape=jax.ShapeDtypeStruct((m, n), jnp.float32),
        compiler_params=pltpu.CompilerParams(
            dimension_semantics=("arbitrary",),
        ),
    )(hidden, pos_emb, w_bf16, bias_2d)


def kernel(tokens, token_emb, pos_emb, W, b):
    batch, seq = tokens.shape
    n = W.shape[1]
    idx = tokens.reshape(batch * seq).astype(jnp.int32)
    hidden = _sc_gather(token_emb, idx)
    out = _tc_mlp(hidden, pos_emb, W.astype(jnp.bfloat16), b.reshape(1, n), seq)
    return out.reshape(batch, seq, n)
